# Initial kernel scaffold; baseline (speedup 1.0000x reference)
import jax
import jax.numpy as jnp
from jax.experimental import pallas as pl


def kernel(x, segment_ids, tok_emb, seg_emb, pos_emb):
    raise NotImplementedError("write your pallas kernel here")


# speedup vs baseline: 5.8759x; 5.8759x over previous
import jax
import jax.numpy as jnp
from jax import lax
from jax.experimental import pallas as pl
from jax.experimental.pallas import tpu as pltpu, tpu_sc as plsc

VOCAB = 100000
D = 128
SEQ = 200
BATCH = 1024
N = BATCH * SEQ
NC = 2
NS = 16
NW = NC * NS
ROWS_W = N // NW
CHUNK = 256
NCHUNK = ROWS_W // CHUNK
IR = ROWS_W // 128


def _body(tok_emb, add_tab, tok_idx, add_idx, out,
          idx_v, aidx_v, rows_v,
          sg0, sg1, sa0, sa1, sw0, sw1):
    wid = lax.axis_index("s") * NC + lax.axis_index("c")
    row0 = wid * ROWS_W

    pltpu.sync_copy(tok_idx.at[wid], idx_v)
    pltpu.sync_copy(add_idx.at[wid], aidx_v)

    gsems = (sg0, sg1)
    asems = (sa0, sa1)
    wsems = (sw0, sw1)

    def start_tok(g, buf):
        for h in range(2):
            pltpu.async_copy(
                tok_emb.at[idx_v.at[2 * g + h]],
                rows_v.at[buf].at[pl.ds(h * 128, 128)],
                gsems[buf])

    def wait_tok(buf):
        for h in range(2):
            pltpu.make_async_copy(
                tok_emb.at[idx_v.at[0]],
                rows_v.at[buf].at[pl.ds(0, 128)],
                gsems[buf]).wait()

    def start_add(g, buf):
        for h in range(2):
            pltpu.async_copy(
                add_tab.at[aidx_v.at[2 * g + h]],
                rows_v.at[buf].at[pl.ds(h * 128, 128)],
                asems[buf], add=True)

    def wait_add(buf):
        for h in range(2):
            pltpu.make_async_copy(
                add_tab.at[aidx_v.at[0]],
                rows_v.at[buf].at[pl.ds(0, 128)],
                asems[buf]).wait()

    def start_write(g, buf):
        pltpu.async_copy(
            rows_v.at[buf],
            out.at[pl.ds(row0 + g * CHUNK, CHUNK)],
            wsems[buf])

    def wait_write(buf):
        pltpu.make_async_copy(
            rows_v.at[buf],
            out.at[pl.ds(0, CHUNK)],
            wsems[buf]).wait()

    start_tok(0, 0)
    for g in range(NCHUNK):
        buf = g % 2
        if g + 1 < NCHUNK:
            nbuf = (g + 1) % 2
            if g >= 1:
                wait_write(nbuf)
            start_tok(g + 1, nbuf)
        wait_tok(buf)
        start_add(g, buf)
        wait_add(buf)
        start_write(g, buf)
    wait_write((NCHUNK - 2) % 2)
    wait_write((NCHUNK - 1) % 2)


def kernel(x, segment_ids, tok_emb, seg_emb, pos_emb):
    tok_idx = x.astype(jnp.int32).reshape(NW, IR, 128)
    pos = jnp.arange(SEQ, dtype=jnp.int32)
    add_idx = (segment_ids.astype(jnp.int32) * SEQ + pos[None, :]).reshape(NW, IR, 128)
    add_tab = (seg_emb[:, None, :] + pos_emb[None, :SEQ, :]).reshape(2 * SEQ, D)

    mesh = plsc.VectorSubcoreMesh(core_axis_name="c", subcore_axis_name="s")
    out = pl.kernel(
        _body,
        out_type=jax.ShapeDtypeStruct((N, D), jnp.float32),
        mesh=mesh,
        scratch_types=[
            pltpu.VMEM((IR, 128), jnp.int32),
            pltpu.VMEM((IR, 128), jnp.int32),
            pltpu.VMEM((2, CHUNK, D), jnp.float32),
            pltpu.SemaphoreType.DMA,
            pltpu.SemaphoreType.DMA,
            pltpu.SemaphoreType.DMA,
            pltpu.SemaphoreType.DMA,
            pltpu.SemaphoreType.DMA,
            pltpu.SemaphoreType.DMA,
        ],
    )(tok_emb, add_tab, tok_idx, add_idx)
    return out.reshape(BATCH, SEQ, D)


# speedup vs baseline: 5.8964x; 1.0035x over previous
import jax
import jax.numpy as jnp
from jax import lax
from jax.experimental import pallas as pl
from jax.experimental.pallas import tpu as pltpu, tpu_sc as plsc

VOCAB = 100000
D = 128
SEQ = 200
BATCH = 1024
N = BATCH * SEQ
NC = 2
NS = 16
NW = NC * NS
ROWS_W = N // NW
CHUNK = 128
NCHUNK = ROWS_W // CHUNK
NBUF = 4
IR = ROWS_W // 128


def _body(tok_emb, add_tab, tok_idx, add_idx, out,
          idx_v, aidx_v, rows_v, *sems):
    wid = lax.axis_index("s") * NC + lax.axis_index("c")
    row0 = wid * ROWS_W

    pltpu.sync_copy(tok_idx.at[wid], idx_v)
    pltpu.sync_copy(add_idx.at[wid], aidx_v)

    gsems = sems[0:NBUF]
    asems = sems[NBUF:2 * NBUF]
    wsems = sems[2 * NBUF:3 * NBUF]

    def start_tok(g, buf):
        pltpu.async_copy(tok_emb.at[idx_v.at[g]], rows_v.at[buf], gsems[buf])

    def wait_tok(buf):
        pltpu.make_async_copy(
            tok_emb.at[idx_v.at[0]], rows_v.at[buf], gsems[buf]).wait()

    def start_add(g, buf):
        pltpu.async_copy(add_tab.at[aidx_v.at[g]], rows_v.at[buf],
                         asems[buf], add=True)

    def wait_add(buf):
        pltpu.make_async_copy(
            add_tab.at[aidx_v.at[0]], rows_v.at[buf], asems[buf]).wait()

    def start_write(g, buf):
        pltpu.async_copy(rows_v.at[buf],
                         out.at[pl.ds(row0 + g * CHUNK, CHUNK)], wsems[buf])

    def wait_write(buf):
        pltpu.make_async_copy(
            rows_v.at[buf], out.at[pl.ds(0, CHUNK)], wsems[buf]).wait()

    for g in range(NCHUNK + 2):
        if g < NCHUNK:
            buf = g % NBUF
            if g >= NBUF:
                wait_write(buf)
            start_tok(g, buf)
        if 0 <= g - 1 < NCHUNK:
            b = (g - 1) % NBUF
            wait_tok(b)
            start_add(g - 1, b)
        if 0 <= g - 2 < NCHUNK:
            b = (g - 2) % NBUF
            wait_add(b)
            start_write(g - 2, b)
    for g in range(NCHUNK - NBUF, NCHUNK):
        wait_write(g % NBUF)


def kernel(x, segment_ids, tok_emb, seg_emb, pos_emb):
    tok_idx = x.astype(jnp.int32).reshape(NW, IR, 128)
    pos = jnp.arange(SEQ, dtype=jnp.int32)
    add_idx = (segment_ids.astype(jnp.int32) * SEQ + pos[None, :]).reshape(NW, IR, 128)
    add_tab = (seg_emb[:, None, :] + pos_emb[None, :SEQ, :]).reshape(2 * SEQ, D)

    mesh = plsc.VectorSubcoreMesh(core_axis_name="c", subcore_axis_name="s")
    out = pl.kernel(
        _body,
        out_type=jax.ShapeDtypeStruct((N, D), jnp.float32),
        mesh=mesh,
        scratch_types=[
            pltpu.VMEM((IR, 128), jnp.int32),
            pltpu.VMEM((IR, 128), jnp.int32),
            pltpu.VMEM((NBUF, CHUNK, D), jnp.float32),
        ] + [pltpu.SemaphoreType.DMA] * (3 * NBUF),
    )(tok_emb, add_tab, tok_idx, add_idx)
    return out.reshape(BATCH, SEQ, D)


# speedup vs baseline: 13.7819x; 2.3373x over previous
import jax
import jax.numpy as jnp
from jax import lax
from jax.experimental import pallas as pl
from jax.experimental.pallas import tpu as pltpu, tpu_sc as plsc

VOCAB = 100000
D = 128
SEQ = 200
BATCH = 1024
N = BATCH * SEQ
NC = 2
NS = 16
NW = NC * NS
ROWS_W = N // NW
CHUNK = 128
NCHUNK = ROWS_W // CHUNK
NBUF = 4
IR = ROWS_W // 128


def _body(tok_emb, add_tab, tok_idx, add_idx, out,
          idx_v, aidx_v, rows_v, atab_s, *sems):
    sid = lax.axis_index("s")
    wid = sid * NC + lax.axis_index("c")
    row0 = wid * ROWS_W

    @pl.when(sid == 0)
    def _():
        pltpu.sync_copy(add_tab, atab_s)

    pltpu.sync_copy(tok_idx.at[wid], idx_v)
    pltpu.sync_copy(add_idx.at[wid], aidx_v)
    plsc.subcore_barrier()

    gsems = sems[0:NBUF]
    asems = sems[NBUF:2 * NBUF]
    wsems = sems[2 * NBUF:3 * NBUF]

    def start_tok(g, buf):
        pltpu.async_copy(tok_emb.at[idx_v.at[g]], rows_v.at[buf], gsems[buf])

    def wait_tok(buf):
        pltpu.make_async_copy(
            tok_emb.at[idx_v.at[0]], rows_v.at[buf], gsems[buf]).wait()

    def start_add(g, buf):
        pltpu.async_copy(atab_s.at[aidx_v.at[g]], rows_v.at[buf],
                         asems[buf], add=True)

    def wait_add(buf):
        pltpu.make_async_copy(
            atab_s.at[aidx_v.at[0]], rows_v.at[buf], asems[buf]).wait()

    def start_write(g, buf):
        pltpu.async_copy(rows_v.at[buf],
                         out.at[pl.ds(row0 + g * CHUNK, CHUNK)], wsems[buf])

    def wait_write(buf):
        pltpu.make_async_copy(
            rows_v.at[buf], out.at[pl.ds(0, CHUNK)], wsems[buf]).wait()

    for g in range(NCHUNK + 2):
        if g < NCHUNK:
            buf = g % NBUF
            if g >= NBUF:
                wait_write(buf)
            start_tok(g, buf)
        if 0 <= g - 1 < NCHUNK:
            b = (g - 1) % NBUF
            wait_tok(b)
            start_add(g - 1, b)
        if 0 <= g - 2 < NCHUNK:
            b = (g - 2) % NBUF
            wait_add(b)
            start_write(g - 2, b)
    for g in range(NCHUNK - NBUF, NCHUNK):
        wait_write(g % NBUF)


def kernel(x, segment_ids, tok_emb, seg_emb, pos_emb):
    tok_idx = x.astype(jnp.int32).reshape(NW, IR, 128)
    pos = jnp.arange(SEQ, dtype=jnp.int32)
    add_idx = (segment_ids.astype(jnp.int32) * SEQ + pos[None, :]).reshape(NW, IR, 128)
    add_tab = (seg_emb[:, None, :] + pos_emb[None, :SEQ, :]).reshape(2 * SEQ, D)

    mesh = plsc.VectorSubcoreMesh(core_axis_name="c", subcore_axis_name="s")
    out = pl.kernel(
        _body,
        out_type=jax.ShapeDtypeStruct((N, D), jnp.float32),
        mesh=mesh,
        scratch_types=[
            pltpu.VMEM((IR, 128), jnp.int32),
            pltpu.VMEM((IR, 128), jnp.int32),
            pltpu.VMEM((NBUF, CHUNK, D), jnp.float32),
            pltpu.VMEM_SHARED((2 * SEQ, D), jnp.float32),
        ] + [pltpu.SemaphoreType.DMA] * (3 * NBUF),
    )(tok_emb, add_tab, tok_idx, add_idx)
    return out.reshape(BATCH, SEQ, D)


# speedup vs baseline: 13.9002x; 1.0086x over previous
import jax
import jax.numpy as jnp
from jax import lax
from jax.experimental import pallas as pl
from jax.experimental.pallas import tpu as pltpu, tpu_sc as plsc

VOCAB = 100000
D = 128
SEQ = 200
BATCH = 1024
N = BATCH * SEQ
NC = 2
NS = 16
NW = NC * NS
ROWS_W = N // NW
CHUNK = 256
NCHUNK = ROWS_W // CHUNK
NBUF = 3
IR = ROWS_W // 128
IPC = CHUNK // 128


def _body(tok_emb, add_tab, tok_idx, add_idx, out,
          idx_v, aidx_v, rows_v, atab_s, *sems):
    sid = lax.axis_index("s")
    wid = sid * NC + lax.axis_index("c")
    row0 = wid * ROWS_W

    psem = sems[3 * NBUF]
    pltpu.async_copy(tok_idx.at[wid], idx_v, psem)
    pltpu.async_copy(add_idx.at[wid], aidx_v, psem)
    @pl.when(sid == 0)
    def _():
        pltpu.sync_copy(add_tab, atab_s)
    pltpu.make_async_copy(tok_idx.at[wid], idx_v, psem).wait()
    pltpu.make_async_copy(add_idx.at[wid], aidx_v, psem).wait()
    plsc.subcore_barrier()

    gsems = sems[0:NBUF]
    asems = sems[NBUF:2 * NBUF]
    wsems = sems[2 * NBUF:3 * NBUF]

    def start_tok(g, buf):
        for h in range(IPC):
            pltpu.async_copy(tok_emb.at[idx_v.at[g * IPC + h]],
                             rows_v.at[buf].at[pl.ds(h * 128, 128)],
                             gsems[buf])

    def wait_tok(buf):
        for h in range(IPC):
            pltpu.make_async_copy(
                tok_emb.at[idx_v.at[0]],
                rows_v.at[buf].at[pl.ds(0, 128)], gsems[buf]).wait()

    def start_add(g, buf):
        for h in range(IPC):
            pltpu.async_copy(atab_s.at[aidx_v.at[g * IPC + h]],
                             rows_v.at[buf].at[pl.ds(h * 128, 128)],
                             asems[buf], add=True)

    def wait_add(buf):
        for h in range(IPC):
            pltpu.make_async_copy(
                atab_s.at[aidx_v.at[0]],
                rows_v.at[buf].at[pl.ds(0, 128)], asems[buf]).wait()

    def start_write(g, buf):
        pltpu.async_copy(rows_v.at[buf],
                         out.at[pl.ds(row0 + g * CHUNK, CHUNK)], wsems[buf])

    def wait_write(buf):
        pltpu.make_async_copy(
            rows_v.at[buf], out.at[pl.ds(0, CHUNK)], wsems[buf]).wait()

    for g in range(NCHUNK + 2):
        if g < NCHUNK:
            buf = g % NBUF
            if g >= NBUF:
                wait_write(buf)
            start_tok(g, buf)
        if 0 <= g - 1 < NCHUNK:
            b = (g - 1) % NBUF
            wait_tok(b)
            start_add(g - 1, b)
        if 0 <= g - 2 < NCHUNK:
            b = (g - 2) % NBUF
            wait_add(b)
            start_write(g - 2, b)
    for g in range(NCHUNK - NBUF, NCHUNK):
        wait_write(g % NBUF)


def kernel(x, segment_ids, tok_emb, seg_emb, pos_emb):
    tok_idx = x.astype(jnp.int32).reshape(NW, IR, 128)
    pos = jnp.arange(SEQ, dtype=jnp.int32)
    add_idx = (segment_ids.astype(jnp.int32) * SEQ + pos[None, :]).reshape(NW, IR, 128)
    add_tab = (seg_emb[:, None, :] + pos_emb[None, :SEQ, :]).reshape(2 * SEQ, D)

    mesh = plsc.VectorSubcoreMesh(core_axis_name="c", subcore_axis_name="s")
    out = pl.kernel(
        _body,
        out_type=jax.ShapeDtypeStruct((N, D), jnp.float32),
        mesh=mesh,
        scratch_types=[
            pltpu.VMEM((IR, 128), jnp.int32),
            pltpu.VMEM((IR, 128), jnp.int32),
            pltpu.VMEM((NBUF, CHUNK, D), jnp.float32),
            pltpu.VMEM_SHARED((2 * SEQ, D), jnp.float32),
        ] + [pltpu.SemaphoreType.DMA] * (3 * NBUF + 1),
    )(tok_emb, add_tab, tok_idx, add_idx)
    return out.reshape(BATCH, SEQ, D)
